# baseline (speedup 1.0000x reference)
import functools

import numpy as np
import jax
import jax.numpy as jnp
from jax import lax
from jax.experimental import pallas as pl
from jax.experimental.pallas import tpu as pltpu
from jax.experimental.pallas import tpu_sc as plsc


def _positional_embedding_np(seq_len, d_model):
    position = np.arange(seq_len, dtype=np.float32)[:, None]
    div_term = np.exp(
        np.arange(0, d_model, 2, dtype=np.float32) * -(np.log(10000.0) / d_model))
    pe = np.zeros((seq_len, d_model), dtype=np.float32)
    pe[:, 0::2] = np.sin(position * div_term)
    pe[:, 1::2] = np.cos(position * div_term)
    return pe


def _sc_gather_body(nt, et, at, idx_all,
                    o_n, o_e,
                    iv, rv0, rv1, rv2, rv3, gsem, wsem):
    rpw = iv.shape[1]
    h = rv0.shape[1]
    wid = lax.axis_index("s") * 2 + lax.axis_index("c")
    base = wid * rpw

    pltpu.sync_copy(idx_all.at[wid], iv)
    g1 = pltpu.async_copy(et.at[iv.at[1]], rv1, gsem)
    g2 = pltpu.async_copy(et.at[iv.at[2]], rv2, gsem)
    g3 = pltpu.async_copy(at.at[iv.at[3]], rv3, gsem)
    g0 = pltpu.async_copy(nt.at[iv.at[0]], rv0, gsem)
    g1.wait()
    g2.wait()
    g3.wait()

    def add_rows(lo, n_rows):
        def add_row(r, carry):
            for c in range(h // 16):
                s = pl.ds(c * 16, 16)
                rv1[r, s] = (rv1[r, s] + rv2[r, s]) + rv3[r, s]
            return carry
        lax.fori_loop(lo, lo + n_rows, add_row, 0)

    half = rpw // 2
    add_rows(0, half)
    w1a = pltpu.async_copy(rv1.at[pl.ds(0, half)],
                           o_e.at[pl.ds(base, half)], wsem)
    add_rows(half, rpw - half)
    w1b = pltpu.async_copy(rv1.at[pl.ds(half, rpw - half)],
                           o_e.at[pl.ds(base + half, rpw - half)], wsem)
    g0.wait()
    w0 = pltpu.async_copy(rv0, o_n.at[pl.ds(base, rpw)], wsem)
    w1a.wait()
    w1b.wait()
    w0.wait()


def _sc_gather(node_table, eidx_table, eattr_table, idx_n, idx_1, idx_2, idx_a):
    n = idx_n.shape[0]
    h = node_table.shape[1]
    nw = 32
    rpw = n // nw
    idx_all = jnp.stack(
        [idx_n.reshape(nw, rpw), idx_1.reshape(nw, rpw),
         idx_2.reshape(nw, rpw), idx_a.reshape(nw, rpw)], axis=1)
    mesh = plsc.VectorSubcoreMesh(core_axis_name="c", subcore_axis_name="s")
    row_t = jax.ShapeDtypeStruct((n, h), jnp.float32)
    f = pl.kernel(
        _sc_gather_body,
        out_type=[row_t, row_t],
        mesh=mesh,
        scratch_types=(
            [pltpu.VMEM((4, rpw), jnp.int32)]
            + [pltpu.VMEM((rpw, h), jnp.float32) for _ in range(4)]
            + [pltpu.SemaphoreType.DMA, pltpu.SemaphoreType.DMA]
        ),
    )
    return f(node_table, eidx_table, eattr_table, idx_all)


def _dotT(u, w):
    return lax.dot_general(u, w, (((1,), (1,)), ((), ())),
                           preferred_element_type=jnp.float32)


def _ln(x, g, b, eps=1e-5):
    mu = jnp.mean(x, axis=-1, keepdims=True)
    d = x - mu
    var = jnp.mean(d * d, axis=-1, keepdims=True)
    return d * lax.rsqrt(var + eps) * g + b


def _tc_body(xg, adj, es, pe, ones_c, W_l, b_l, W_r, W_proj, b_proj,
             ln_g, ln_b, gW1, gb1, gW2, gb2, out):
    x = xg[...] + pe[...]
    a = adj[...]
    agg = lax.dot_general(a, x.astype(jnp.bfloat16), (((0,), (0,)), ((), ())),
                          preferred_element_type=jnp.float32)
    cnt = lax.dot_general(a, ones_c[...], (((0,), (0,)), ((), ())),
                          preferred_element_type=jnp.float32)
    agg = agg / jnp.maximum(cnt, 1.0)
    h = _dotT(agg, W_l[...]) + b_l[...] + _dotT(x, W_r[...])
    h = _dotT(h, W_proj[...]) + b_proj[...]
    h = _ln(h, ln_g[...], ln_b[...])
    r = jnp.maximum(_dotT(h, gW1[...]) + gb1[...], 0.0)
    z = jnp.sum(r * gW2[...], axis=-1, keepdims=True) + gb2[...]
    g = 1.0 / (1.0 + jnp.exp(-z))
    h = h * g
    out[0] = _ln(h + es[...], ln_g[...], ln_b[...])


def _tc_forward(xg, adj, es, pe, weights, interpret=False):
    B, S, _ = adj.shape
    H = xg.shape[-1]
    row_spec = pl.BlockSpec((None, S, H), lambda b: (b, 0, 0))
    full2d = lambda arr: pl.BlockSpec(arr.shape, lambda b: (0,) * arr.ndim)
    ones_c = jnp.ones((S, 1), jnp.bfloat16)
    in_specs = [
        row_spec,
        pl.BlockSpec((None, S, S), lambda b: (b, 0, 0)),
        row_spec,
        full2d(pe), full2d(ones_c),
    ] + [full2d(w) for w in weights]
    out = pl.pallas_call(
        _tc_body,
        grid=(B,),
        in_specs=in_specs,
        out_specs=pl.BlockSpec((1, S, H), lambda b: (b, 0, 0)),
        out_shape=jax.ShapeDtypeStruct((B, S, H), jnp.float32),
        interpret=interpret,
    )(xg.reshape(B, S, H), adj, es.reshape(B, S, H), pe, ones_c, *weights)
    return out


def kernel(nodes, edge_indices, edge_attr, adj_matrices, node_table,
           eidx_table, eattr_table, W_l, b_l, W_r, W_proj, b_proj,
           ln_g, ln_b, gW1, gb1, gW2, gb2):
    B, S = nodes.shape
    H = node_table.shape[1]
    idx_n = nodes.reshape(-1)
    idx_1 = edge_indices[..., 0].reshape(-1)
    idx_2 = edge_indices[..., 1].reshape(-1)
    idx_a = edge_attr.reshape(-1)
    xg, es = _sc_gather(node_table, eidx_table, eattr_table,
                        idx_n, idx_1, idx_2, idx_a)
    pe = jnp.asarray(_positional_embedding_np(S, H))
    weights = (W_l, b_l.reshape(1, H), W_r, W_proj, b_proj.reshape(1, H),
               ln_g.reshape(1, H), ln_b.reshape(1, H),
               gW1, gb1.reshape(1, H // 2), gW2, gb2.reshape(1, 1))
    adj_bf = (adj_matrices != 0).astype(jnp.bfloat16)
    return _tc_forward(xg, adj_bf, es, pe, weights)


# speedup vs baseline: 1.0394x; 1.0394x over previous
import functools

import numpy as np
import jax
import jax.numpy as jnp
from jax import lax
from jax.experimental import pallas as pl
from jax.experimental.pallas import tpu as pltpu
from jax.experimental.pallas import tpu_sc as plsc


def _positional_embedding_np(seq_len, d_model):
    position = np.arange(seq_len, dtype=np.float32)[:, None]
    div_term = np.exp(
        np.arange(0, d_model, 2, dtype=np.float32) * -(np.log(10000.0) / d_model))
    pe = np.zeros((seq_len, d_model), dtype=np.float32)
    pe[:, 0::2] = np.sin(position * div_term)
    pe[:, 1::2] = np.cos(position * div_term)
    return pe


def _sc_gather_body(nt, et, at, idx_all,
                    o_n, o_e,
                    iv, rv0, rv1, rv2, rv3, gsem, wsem):
    rpw = iv.shape[1]
    h = rv0.shape[1]
    wid = lax.axis_index("s") * 2 + lax.axis_index("c")
    base = wid * rpw

    pltpu.sync_copy(idx_all.at[wid], iv)
    g1 = pltpu.async_copy(et.at[iv.at[1]], rv1, gsem)
    g2 = pltpu.async_copy(et.at[iv.at[2]], rv2, gsem)
    g3 = pltpu.async_copy(at.at[iv.at[3]], rv3, gsem)
    g0 = pltpu.async_copy(nt.at[iv.at[0]], rv0, gsem)
    g1.wait()
    g2.wait()
    g3.wait()

    def add_rows(lo, n_rows):
        def add_row(r, carry):
            for c in range(h // 16):
                s = pl.ds(c * 16, 16)
                rv1[r, s] = (rv1[r, s] + rv2[r, s]) + rv3[r, s]
            return carry
        lax.fori_loop(lo, lo + n_rows, add_row, 0)

    half = rpw // 2
    add_rows(0, half)
    w1a = pltpu.async_copy(rv1.at[pl.ds(0, half)],
                           o_e.at[pl.ds(base, half)], wsem)
    add_rows(half, rpw - half)
    w1b = pltpu.async_copy(rv1.at[pl.ds(half, rpw - half)],
                           o_e.at[pl.ds(base + half, rpw - half)], wsem)
    g0.wait()
    w0 = pltpu.async_copy(rv0, o_n.at[pl.ds(base, rpw)], wsem)
    w1a.wait()
    w1b.wait()
    w0.wait()


def _sc_gather(node_table, eidx_table, eattr_table, idx_n, idx_1, idx_2, idx_a):
    n = idx_n.shape[0]
    h = node_table.shape[1]
    nw = 32
    rpw = n // nw
    idx_all = jnp.stack(
        [idx_n.reshape(nw, rpw), idx_1.reshape(nw, rpw),
         idx_2.reshape(nw, rpw), idx_a.reshape(nw, rpw)], axis=1)
    mesh = plsc.VectorSubcoreMesh(core_axis_name="c", subcore_axis_name="s")
    row_t = jax.ShapeDtypeStruct((n, h), jnp.float32)
    f = pl.kernel(
        _sc_gather_body,
        out_type=[row_t, row_t],
        mesh=mesh,
        scratch_types=(
            [pltpu.VMEM((4, rpw), jnp.int32)]
            + [pltpu.VMEM((rpw, h), jnp.float32) for _ in range(4)]
            + [pltpu.SemaphoreType.DMA, pltpu.SemaphoreType.DMA]
        ),
    )
    return f(node_table, eidx_table, eattr_table, idx_all)


def _dotT(u, w):
    return lax.dot_general(u, w, (((1,), (1,)), ((), ())),
                           preferred_element_type=jnp.float32)


def _ln(x, g, b, eps=1e-5):
    mu = jnp.mean(x, axis=-1, keepdims=True)
    d = x - mu
    var = jnp.mean(d * d, axis=-1, keepdims=True)
    return d * lax.rsqrt(var + eps) * g + b


def _tc_body(xg, adj, es, pe, ones_c, W_l, b_l, W_r, W_proj, b_proj,
             ln_g, ln_b, gW1, gb1, gW2, gb2, out):
    B = adj.shape[0]
    for b in range(B):
        x = xg[b] + pe[...]
        a = (adj[b] != 0).astype(jnp.bfloat16)
        agg = lax.dot_general(a, x.astype(jnp.bfloat16),
                              (((0,), (0,)), ((), ())),
                              preferred_element_type=jnp.float32)
        cnt = lax.dot_general(a, ones_c[...], (((0,), (0,)), ((), ())),
                              preferred_element_type=jnp.float32)
        agg = agg / jnp.maximum(cnt, 1.0)
        h = _dotT(agg, W_l[...]) + b_l[...] + _dotT(x, W_r[...])
        h = _dotT(h, W_proj[...]) + b_proj[...]
        h = _ln(h, ln_g[...], ln_b[...])
        r = jnp.maximum(_dotT(h, gW1[...]) + gb1[...], 0.0)
        z = jnp.sum(r * gW2[...], axis=-1, keepdims=True) + gb2[...]
        g = 1.0 / (1.0 + jnp.exp(-z))
        h = h * g
        out[b] = _ln(h + es[b], ln_g[...], ln_b[...])


def _tc_forward(xg, adj, es, pe, weights, interpret=False):
    B, S, _ = adj.shape
    H = xg.shape[-1]
    ones_c = jnp.ones((S, 1), jnp.bfloat16)
    out = pl.pallas_call(
        _tc_body,
        out_shape=jax.ShapeDtypeStruct((B, S, H), jnp.float32),
        interpret=interpret,
    )(xg.reshape(B, S, H), adj, es.reshape(B, S, H), pe, ones_c, *weights)
    return out


def kernel(nodes, edge_indices, edge_attr, adj_matrices, node_table,
           eidx_table, eattr_table, W_l, b_l, W_r, W_proj, b_proj,
           ln_g, ln_b, gW1, gb1, gW2, gb2):
    B, S = nodes.shape
    H = node_table.shape[1]
    idx_n = nodes.reshape(-1)
    idx_1 = edge_indices[..., 0].reshape(-1)
    idx_2 = edge_indices[..., 1].reshape(-1)
    idx_a = edge_attr.reshape(-1)
    xg, es = _sc_gather(node_table, eidx_table, eattr_table,
                        idx_n, idx_1, idx_2, idx_a)
    pe = jnp.asarray(_positional_embedding_np(S, H))
    weights = (W_l, b_l.reshape(1, H), W_r, W_proj, b_proj.reshape(1, H),
               ln_g.reshape(1, H), ln_b.reshape(1, H),
               gW1, gb1.reshape(1, H // 2), gW2, gb2.reshape(1, 1))
    return _tc_forward(xg, adj_matrices, es, pe, weights)


# speedup vs baseline: 1.0430x; 1.0034x over previous
import numpy as np
import jax
import jax.numpy as jnp
from jax import lax
from jax.experimental import pallas as pl
from jax.experimental.pallas import tpu as pltpu
from jax.experimental.pallas import tpu_sc as plsc


def _positional_embedding_np(seq_len, d_model):
    position = np.arange(seq_len, dtype=np.float32)[:, None]
    div_term = np.exp(
        np.arange(0, d_model, 2, dtype=np.float32) * -(np.log(10000.0) / d_model))
    pe = np.zeros((seq_len, d_model), dtype=np.float32)
    pe[:, 0::2] = np.sin(position * div_term)
    pe[:, 1::2] = np.cos(position * div_term)
    return pe


def _sc_gather_body(nt, et, at, idx_all,
                    o_n, o_e,
                    iv, rv0, rv1, rv2, rv3, gsem, wsem):
    rpw = iv.shape[1]
    h = rv0.shape[1]
    wid = lax.axis_index("s") * 2 + lax.axis_index("c")
    base = wid * rpw

    pltpu.sync_copy(idx_all.at[wid], iv)
    g1 = pltpu.async_copy(et.at[iv.at[1]], rv1, gsem)
    g2 = pltpu.async_copy(et.at[iv.at[2]], rv2, gsem)
    g3 = pltpu.async_copy(at.at[iv.at[3]], rv3, gsem)
    g0 = pltpu.async_copy(nt.at[iv.at[0]], rv0, gsem)
    g1.wait()
    g2.wait()
    g3.wait()

    def add_rows(lo, n_rows):
        def add_row(r, carry):
            for c in range(h // 16):
                s = pl.ds(c * 16, 16)
                rv1[r, s] = (rv1[r, s] + rv2[r, s]) + rv3[r, s]
            return carry
        lax.fori_loop(lo, lo + n_rows, add_row, 0)

    half = rpw // 2
    add_rows(0, half)
    w1a = pltpu.async_copy(rv1.at[pl.ds(0, half)],
                           o_e.at[pl.ds(base, half)], wsem)
    add_rows(half, rpw - half)
    w1b = pltpu.async_copy(rv1.at[pl.ds(half, rpw - half)],
                           o_e.at[pl.ds(base + half, rpw - half)], wsem)
    g0.wait()
    w0 = pltpu.async_copy(rv0, o_n.at[pl.ds(base, rpw)], wsem)
    w1a.wait()
    w1b.wait()
    w0.wait()


def _sc_gather(node_table, eidx_table, eattr_table, idx_n, idx_1, idx_2, idx_a):
    n = idx_n.shape[0]
    h = node_table.shape[1]
    nw = 32
    rpw = n // nw
    idx_all = jnp.stack(
        [idx_n.reshape(nw, rpw), idx_1.reshape(nw, rpw),
         idx_2.reshape(nw, rpw), idx_a.reshape(nw, rpw)], axis=1)
    mesh = plsc.VectorSubcoreMesh(core_axis_name="c", subcore_axis_name="s")
    row_t = jax.ShapeDtypeStruct((n, h), jnp.float32)
    f = pl.kernel(
        _sc_gather_body,
        out_type=[row_t, row_t],
        mesh=mesh,
        scratch_types=(
            [pltpu.VMEM((4, rpw), jnp.int32)]
            + [pltpu.VMEM((rpw, h), jnp.float32) for _ in range(4)]
            + [pltpu.SemaphoreType.DMA, pltpu.SemaphoreType.DMA]
        ),
    )
    return f(node_table, eidx_table, eattr_table, idx_all)


def _dotT(u, w):
    return lax.dot_general(u, w, (((1,), (1,)), ((), ())),
                           preferred_element_type=jnp.float32)


def _ln(x, g, b, eps=1e-5):
    mu = jnp.mean(x, axis=-1, keepdims=True)
    d = x - mu
    var = jnp.mean(d * d, axis=-1, keepdims=True)
    return d * lax.rsqrt(var + eps) * g + b


def _tc_body(xg, adj, es, pe, ones_c, W_l, b_l, W_r, W_proj, b_proj,
             ln_g, ln_b, gW1, gb1, gW2, gb2, out):
    B = adj.shape[0]
    for b in range(B):
        x = xg[b] + pe[...]
        a = (adj[b] != 0).astype(jnp.bfloat16)
        agg = lax.dot_general(a, x.astype(jnp.bfloat16),
                              (((0,), (0,)), ((), ())),
                              preferred_element_type=jnp.float32)
        cnt = lax.dot_general(a, ones_c[...], (((0,), (0,)), ((), ())),
                              preferred_element_type=jnp.float32)
        agg = agg / jnp.maximum(cnt, 1.0)
        h = _dotT(agg, W_l[...]) + b_l[...] + _dotT(x, W_r[...])
        h = _dotT(h, W_proj[...]) + b_proj[...]
        h = _ln(h, ln_g[...], ln_b[...])
        r = jnp.maximum(_dotT(h, gW1[...]) + gb1[...], 0.0)
        z = jnp.sum(r * gW2[...], axis=-1, keepdims=True) + gb2[...]
        g = 1.0 / (1.0 + jnp.exp(-z))
        h = h * g
        out[b] = _ln(h + es[b], ln_g[...], ln_b[...])


def _tc_forward(xg, adj, es, pe, weights):
    B, S, _ = adj.shape
    H = xg.shape[-1]
    ones_c = jnp.ones((S, 1), jnp.bfloat16)
    out = pl.pallas_call(
        _tc_body,
        out_shape=jax.ShapeDtypeStruct((B, S, H), jnp.float32),
    )(xg.reshape(B, S, H), adj, es.reshape(B, S, H), pe, ones_c, *weights)
    return out


def kernel(nodes, edge_indices, edge_attr, adj_matrices, node_table,
           eidx_table, eattr_table, W_l, b_l, W_r, W_proj, b_proj,
           ln_g, ln_b, gW1, gb1, gW2, gb2):
    B, S = nodes.shape
    H = node_table.shape[1]
    idx_n = nodes.reshape(-1)
    idx_1 = edge_indices[..., 0].reshape(-1)
    idx_2 = edge_indices[..., 1].reshape(-1)
    idx_a = edge_attr.reshape(-1)
    xg, es = _sc_gather(node_table, eidx_table, eattr_table,
                        idx_n, idx_1, idx_2, idx_a)
    pe = jnp.asarray(_positional_embedding_np(S, H))
    weights = (W_l, b_l.reshape(1, H), W_r, W_proj, b_proj.reshape(1, H),
               ln_g.reshape(1, H), ln_b.reshape(1, H),
               gW1, gb1.reshape(1, H // 2), gW2, gb2.reshape(1, 1))
    return _tc_forward(xg, adj_matrices, es, pe, weights)
